# baseline (speedup 1.0000x reference)
import functools
import math

import jax
import jax.numpy as jnp
from jax import lax
from jax.experimental import pallas as pl
from jax.experimental.pallas import tpu as pltpu
from jax.experimental.pallas import tpu_sc as plsc

NC = 2
NS = 16
NW = NC * NS

CHUNK = 32


def _sc_gather(table, idx):
    B = idx.shape[0]
    V, D = table.shape
    b_per_w = B // NW
    n_chunks = b_per_w // CHUNK
    mesh = plsc.VectorSubcoreMesh(core_axis_name="c", subcore_axis_name="s")

    @functools.partial(
        pl.kernel,
        mesh=mesh,
        out_type=jax.ShapeDtypeStruct((B, D), jnp.float32),
        scratch_types=[
            pltpu.VMEM((b_per_w,), jnp.int32),
            pltpu.VMEM((CHUNK, D), jnp.float32),
            pltpu.VMEM((CHUNK, D), jnp.float32),
            pltpu.SemaphoreType.DMA,
            pltpu.SemaphoreType.DMA,
            pltpu.SemaphoreType.DMA,
            pltpu.SemaphoreType.DMA,
        ],
    )
    def k(table_hbm, idx_hbm, out_hbm, idx_v, buf0, buf1, g0, g1, w0, w1):
        wid = lax.axis_index("s") * NC + lax.axis_index("c")
        base = wid * b_per_w
        pltpu.sync_copy(idx_hbm.at[pl.ds(base, b_per_w)], idx_v)

        def g_copy(c, buf, sem):
            return pltpu.make_async_copy(
                table_hbm.at[idx_v.at[pl.ds(c * CHUNK, CHUNK)]], buf, sem
            )

        def w_copy(c, buf, sem):
            return pltpu.make_async_copy(
                buf, out_hbm.at[pl.ds(base + c * CHUNK, CHUNK)], sem
            )

        g_copy(0, buf0, g0).start()
        g_copy(1, buf1, g1).start()

        @pl.loop(0, n_chunks - 2, step=2)
        def _(c):
            g_copy(c, buf0, g0).wait()
            w_copy(c, buf0, w0).start()
            g_copy(c + 1, buf1, g1).wait()
            w_copy(c + 1, buf1, w1).start()
            w_copy(c, buf0, w0).wait()
            g_copy(c + 2, buf0, g0).start()
            w_copy(c + 1, buf1, w1).wait()
            g_copy(c + 3, buf1, g1).start()

        g_copy(n_chunks - 2, buf0, g0).wait()
        w_copy(n_chunks - 2, buf0, w0).start()
        g_copy(n_chunks - 1, buf1, g1).wait()
        w_copy(n_chunks - 1, buf1, w1).start()
        w_copy(n_chunks - 2, buf0, w0).wait()
        w_copy(n_chunks - 1, buf1, w1).wait()

    return k(table, idx)


NPARTS = 2
BS = 512


def _tc_scale_add_part(g_part, pe, prev_out, S, s0, scale, batch):
    SB, D = g_part.shape
    S_part = SB // batch
    blk0 = s0 // BS

    def body(g_ref, pe_ref, *rest):
        o_ref = rest[-1]
        g3 = g_ref[...].reshape(BS, batch, D)
        o_ref[...] = g3 * scale + pe_ref[...]

    in_specs = [
        pl.BlockSpec((BS * batch, D), lambda i: (i, 0)),
        pl.BlockSpec((BS, 1, D), lambda i: (i + blk0, 0, 0)),
    ]
    args = [g_part, pe]
    aliases = {}
    if prev_out is not None:
        in_specs.append(pl.BlockSpec(memory_space=pl.ANY))
        args.append(prev_out)
        aliases = {2: 0}

    return pl.pallas_call(
        body,
        grid=(S_part // BS,),
        in_specs=in_specs,
        out_specs=pl.BlockSpec((BS, batch, D), lambda i: (i + blk0, 0, 0)),
        out_shape=jax.ShapeDtypeStruct((S, batch, D), jnp.float32),
        input_output_aliases=aliases,
    )(*args)


def kernel(x, emb_table, pe):
    S, B = x.shape
    V, D = emb_table.shape
    scale = math.sqrt(D)
    idx = x.reshape(-1).astype(jnp.int32)
    SB = S * B
    part_rows = SB // NPARTS
    gs = [
        _sc_gather(emb_table, idx[p * part_rows:(p + 1) * part_rows])
        for p in range(NPARTS)
    ]
    out = None
    for p in range(NPARTS):
        s0 = p * (S // NPARTS)
        out = _tc_scale_add_part(gs[p], pe, out, S, s0, scale, B)
    return out


# speedup vs baseline: 1.0100x; 1.0100x over previous
import functools
import math

import jax
import jax.numpy as jnp
from jax import lax
from jax.experimental import pallas as pl
from jax.experimental.pallas import tpu as pltpu
from jax.experimental.pallas import tpu_sc as plsc

NC = 2
NS = 16
NW = NC * NS

CHUNK = 32


def _sc_gather(table, idx):
    B = idx.shape[0]
    V, D = table.shape
    b_per_w = B // NW
    n_chunks = b_per_w // CHUNK
    mesh = plsc.VectorSubcoreMesh(core_axis_name="c", subcore_axis_name="s")

    @functools.partial(
        pl.kernel,
        mesh=mesh,
        out_type=jax.ShapeDtypeStruct((B, D), jnp.float32),
        scratch_types=[
            pltpu.VMEM((b_per_w,), jnp.int32),
            pltpu.VMEM((CHUNK, D), jnp.float32),
            pltpu.VMEM((CHUNK, D), jnp.float32),
            pltpu.SemaphoreType.DMA,
            pltpu.SemaphoreType.DMA,
            pltpu.SemaphoreType.DMA,
            pltpu.SemaphoreType.DMA,
        ],
    )
    def k(table_hbm, idx_hbm, out_hbm, idx_v, buf0, buf1, g0, g1, w0, w1):
        wid = lax.axis_index("s") * NC + lax.axis_index("c")
        base = wid * b_per_w
        pltpu.sync_copy(idx_hbm.at[pl.ds(base, b_per_w)], idx_v)

        def g_copy(c, buf, sem):
            return pltpu.make_async_copy(
                table_hbm.at[idx_v.at[pl.ds(c * CHUNK, CHUNK)]], buf, sem
            )

        def w_copy(c, buf, sem):
            return pltpu.make_async_copy(
                buf, out_hbm.at[pl.ds(base + c * CHUNK, CHUNK)], sem
            )

        g_copy(0, buf0, g0).start()
        g_copy(1, buf1, g1).start()

        @pl.loop(0, n_chunks - 2, step=2)
        def _(c):
            g_copy(c, buf0, g0).wait()
            w_copy(c, buf0, w0).start()
            g_copy(c + 1, buf1, g1).wait()
            w_copy(c + 1, buf1, w1).start()
            w_copy(c, buf0, w0).wait()
            g_copy(c + 2, buf0, g0).start()
            w_copy(c + 1, buf1, w1).wait()
            g_copy(c + 3, buf1, g1).start()

        g_copy(n_chunks - 2, buf0, g0).wait()
        w_copy(n_chunks - 2, buf0, w0).start()
        g_copy(n_chunks - 1, buf1, g1).wait()
        w_copy(n_chunks - 1, buf1, w1).start()
        w_copy(n_chunks - 2, buf0, w0).wait()
        w_copy(n_chunks - 1, buf1, w1).wait()

    return k(table, idx)


BS = 512


def _tc_scale_add(g, pe, S, scale, batch):
    SB, D = g.shape

    def body(g_ref, pe_ref, o_ref):
        g3 = g_ref[...].reshape(BS, batch, D)
        o_ref[...] = g3 * scale + pe_ref[...]

    return pl.pallas_call(
        body,
        grid=(S // BS,),
        in_specs=[
            pl.BlockSpec((BS * batch, D), lambda i: (i, 0)),
            pl.BlockSpec((BS, 1, D), lambda i: (i, 0, 0)),
        ],
        out_specs=pl.BlockSpec((BS, batch, D), lambda i: (i, 0, 0)),
        out_shape=jax.ShapeDtypeStruct((S, batch, D), jnp.float32),
        compiler_params=pltpu.CompilerParams(
            dimension_semantics=("parallel",)
        ),
    )(g, pe)


def kernel(x, emb_table, pe):
    S, B = x.shape
    V, D = emb_table.shape
    idx = x.reshape(-1).astype(jnp.int32)
    g = _sc_gather(emb_table, idx)
    return _tc_scale_add(g, pe, S, math.sqrt(D), B)


# speedup vs baseline: 1.0200x; 1.0099x over previous
import functools
import math

import jax
import jax.numpy as jnp
from jax import lax
from jax.experimental import pallas as pl
from jax.experimental.pallas import tpu as pltpu
from jax.experimental.pallas import tpu_sc as plsc

NC = 2
NS = 16
NW = NC * NS

CHUNK = 16
NBUF = 4


def _sc_gather(table, idx):
    B = idx.shape[0]
    V, D = table.shape
    b_per_w = B // NW
    n_chunks = b_per_w // CHUNK
    mesh = plsc.VectorSubcoreMesh(core_axis_name="c", subcore_axis_name="s")

    @functools.partial(
        pl.kernel,
        mesh=mesh,
        out_type=jax.ShapeDtypeStruct((B, D), jnp.float32),
        scratch_types=[pltpu.VMEM((b_per_w,), jnp.int32)]
        + [pltpu.VMEM((CHUNK, D), jnp.float32)] * NBUF
        + [pltpu.SemaphoreType.DMA] * (2 * NBUF),
    )
    def k(table_hbm, idx_hbm, out_hbm, idx_v, *bufs_sems):
        bufs = bufs_sems[:NBUF]
        gsems = bufs_sems[NBUF:2 * NBUF]
        wsems = bufs_sems[2 * NBUF:]
        wid = lax.axis_index("s") * NC + lax.axis_index("c")
        base = wid * b_per_w
        pltpu.sync_copy(idx_hbm.at[pl.ds(base, b_per_w)], idx_v)

        def g_copy(c, j):
            return pltpu.make_async_copy(
                table_hbm.at[idx_v.at[pl.ds(c * CHUNK, CHUNK)]],
                bufs[j], gsems[j]
            )

        def w_copy(c, j):
            return pltpu.make_async_copy(
                bufs[j], out_hbm.at[pl.ds(base + c * CHUNK, CHUNK)], wsems[j]
            )

        for j in range(NBUF):
            g_copy(j, j).start()

        @pl.loop(0, n_chunks - NBUF, step=NBUF)
        def _(c):
            for j in range(NBUF):
                g_copy(c + j, j).wait()
                w_copy(c + j, j).start()
            for j in range(NBUF):
                w_copy(c + j, j).wait()
                g_copy(c + NBUF + j, j).start()

        for j in range(NBUF):
            g_copy(n_chunks - NBUF + j, j).wait()
            w_copy(n_chunks - NBUF + j, j).start()
        for j in range(NBUF):
            w_copy(n_chunks - NBUF + j, j).wait()

    return k(table, idx)


BS = 512


def _tc_scale_add(g, pe, S, scale, batch):
    SB, D = g.shape

    def body(g_ref, pe_ref, o_ref):
        g3 = g_ref[...].reshape(BS, batch, D)
        o_ref[...] = g3 * scale + pe_ref[...]

    return pl.pallas_call(
        body,
        grid=(S // BS,),
        in_specs=[
            pl.BlockSpec((BS * batch, D), lambda i: (i, 0)),
            pl.BlockSpec((BS, 1, D), lambda i: (i, 0, 0)),
        ],
        out_specs=pl.BlockSpec((BS, batch, D), lambda i: (i, 0, 0)),
        out_shape=jax.ShapeDtypeStruct((S, batch, D), jnp.float32),
        compiler_params=pltpu.CompilerParams(
            dimension_semantics=("parallel",)
        ),
    )(g, pe)


def kernel(x, emb_table, pe):
    S, B = x.shape
    V, D = emb_table.shape
    idx = x.reshape(-1).astype(jnp.int32)
    g = _sc_gather(emb_table, idx)
    return _tc_scale_add(g, pe, S, math.sqrt(D), B)
